# baseline (speedup 1.0000x reference)
import functools

import jax
import jax.numpy as jnp
from jax.experimental import pallas as pl

B, N, L, D = 2, 512, 32, 64
E, K = 16, 2
HT, HN = 4, 4
OUT = 64
DH = D // HT
DHN = OUT // HN
L2 = L // 2
SEQ = B * N

SBLK = 16
TBLK = SBLK * L
SPLIT = 2
TH = TBLK // SPLIT

_NEG = -1e9


def _ln(z, g, b):
    m = jnp.mean(z, axis=-1, keepdims=True)
    v = jnp.mean((z - m) ** 2, axis=-1, keepdims=True)
    return (z - m) * jax.lax.rsqrt(v + 1e-5) * g + b


def _block_kernel(x_ref, bias_ref, onesbd_ref, wq_ref, wk_ref, wv_ref,
                  bq_ref, bk_ref, bv_ref, wo_ref, bo_ref, ln1g_ref, ln1b_ref,
                  wr_ref, br_ref, w1cat_ref, b1cat_ref, w2stack_ref, be2_ref,
                  emat_ref, ln2g_ref, ln2b_ref, wd_ref, bd_ref, out_ref):
    xb = x_ref[...].reshape(TBLK, D)
    f32 = jnp.float32
    bf = jnp.bfloat16
    bias = bias_ref[...]

    q = jnp.dot(xb, wq_ref[...], preferred_element_type=f32) + bq_ref[...]
    k = jnp.dot(xb, wk_ref[...], preferred_element_type=f32) + bk_ref[...]
    v = jnp.dot(xb, wv_ref[...], preferred_element_type=f32) + bv_ref[...]
    vb = v.astype(bf)
    o_parts = []
    for part in range(SPLIT):
        r0 = part * TH
        heads = []
        ps = []
        for h in range(HT):
            qh = q[r0:r0 + TH, h * DH:(h + 1) * DH]
            kh = k[r0:r0 + TH, h * DH:(h + 1) * DH]
            vh = vb[r0:r0 + TH, h * DH:(h + 1) * DH]
            s = jax.lax.dot_general(qh, kh, (((1,), (1,)), ((), ())),
                                    preferred_element_type=f32) + bias
            p = jnp.exp2(s).astype(bf)
            ps.append(p)
            heads.append(jnp.dot(p, vh, preferred_element_type=f32))
        sums = jnp.dot(jnp.concatenate(ps, axis=-1), onesbd_ref[...],
                       preferred_element_type=f32)
        inv = 1.0 / sums
        inv_cat = jnp.concatenate(
            [jnp.broadcast_to(inv[:, h:h + 1], (TH, DH)) for h in range(HT)],
            axis=-1)
        o_parts.append(jnp.concatenate(heads, axis=-1) * inv_cat)
    o = jnp.concatenate(o_parts, axis=0)
    o = jnp.dot(o, wo_ref[...], preferred_element_type=f32) + bo_ref[...]
    h1 = _ln(xb + o, ln1g_ref[...], ln1b_ref[...])

    logits = jnp.dot(h1, wr_ref[...], preferred_element_type=f32) + br_ref[...]
    eiota = jax.lax.broadcasted_iota(jnp.int32, (TBLK, E), 1)
    m1 = jnp.max(logits, axis=-1, keepdims=True)
    i1 = jnp.min(jnp.where(logits == m1, eiota, E), axis=-1, keepdims=True)
    lmask = jnp.where(eiota == i1, -jnp.inf, logits)
    m2 = jnp.max(lmask, axis=-1, keepdims=True)
    i2 = jnp.min(jnp.where(lmask == m2, eiota, E), axis=-1, keepdims=True)
    t = jnp.exp(m2 - m1)
    g1 = 1.0 / (1.0 + t)
    g2 = t / (1.0 + t)
    w16 = jnp.where(eiota == i1, g1, 0.0) + jnp.where(eiota == i2, g2, 0.0)

    hcat = jnp.maximum(
        jnp.dot(h1.astype(bf), w1cat_ref[...].astype(bf),
                preferred_element_type=f32) + b1cat_ref[...], 0.0)
    wexp = jnp.dot(w16.astype(bf), emat_ref[...].astype(bf),
                   preferred_element_type=f32)
    acc = (jnp.dot((hcat * wexp).astype(bf), w2stack_ref[...].astype(bf),
                   preferred_element_type=f32)
           + jnp.dot(w16, be2_ref[...], preferred_element_type=f32))

    h2 = _ln(h1 + acc, ln2g_ref[...], ln2b_ref[...])

    dz = jnp.dot(h2.astype(bf), wd_ref[...].astype(bf),
                 preferred_element_type=f32) + bd_ref[...]
    dz = jnp.maximum(dz, jnp.exp(jnp.minimum(dz, 0.0)) - 1.0)
    out_ref[...] = jnp.max(dz.reshape(TBLK // 2, 2, D), axis=1).reshape(
        SBLK, L2, D)


def _gat_kernel(tc_ref, mask_ref, wg_ref, asrc_ref, adst_ref, hidx_ref,
                ones_ref, out_ref):
    f32 = jnp.float32
    bf = jnp.bfloat16
    hidx = hidx_ref[...]
    zero = jnp.zeros((), bf)
    for l2s in range(2):
      mask = mask_ref[l2s]
      for bb in range(B):
        tc = tc_ref[bb * N:(bb + 1) * N, l2s, 0, :]
        hgf = jnp.dot(tc.astype(bf), wg_ref[...].astype(bf),
                      preferred_element_type=f32)
        hg = hgf.astype(bf)
        s4 = jnp.dot(hgf, asrc_ref[...], preferred_element_type=f32)
        d4 = jnp.dot(hgf, adst_ref[...], preferred_element_type=f32)
        d4t = d4.T
        es = jnp.exp(s4).astype(bf)
        es5 = jnp.exp(0.2 * s4).astype(bf)
        ed = jnp.exp(d4t).astype(bf)
        ed5 = jnp.exp(0.2 * d4t).astype(bf)
        alphas = []
        vstack = []
        for h in range(HN):
            s5c, ec = es5[:, h:h + 1], es[:, h:h + 1]
            edr, ed5r = ed[h:h + 1, :], ed5[h:h + 1, :]
            alphas.append(jnp.maximum(ec * edr, s5c * ed5r) * mask)
            vstack.append(jnp.where(hidx == h, hg, zero))
        p_cat = jnp.concatenate(alphas, axis=-1)
        v_stack = jnp.concatenate(vstack, axis=0)
        go = jnp.dot(p_cat, v_stack, preferred_element_type=f32)
        sums = jnp.dot(p_cat, ones_ref[...], preferred_element_type=f32)
        inv = 1.0 / sums
        inv_cat = jnp.concatenate(
            [jnp.broadcast_to(inv[:, h:h + 1], (N, DHN)) for h in range(HN)],
            axis=-1)
        out_ref[bb, :, l2s, 0, :] = jnp.maximum(go * inv_cat, 0.0)


@functools.partial(jax.jit, static_argnums=())
def kernel(x, adj, Wq, Wk, Wv, bq, bk, bv, Wo, bo, ln1_g, ln1_b, Wr, br,
           We1, be1, We2, be2, ln2_g, ln2_b, Wd, bd, Wg, a_src, a_dst):
    f32 = jnp.float32
    x3 = x.reshape(SEQ, L, D)
    scale = 1.4426950408889634 / (DH ** 0.5)
    wqs = Wq * scale
    bqs = bq * scale
    w1cat = We1.transpose(1, 0, 2).reshape(D, E * D)
    b1cat = be1.reshape(1, E * D)
    w2stack = We2.reshape(E * D, D)
    emat = jnp.kron(jnp.eye(E, dtype=f32), jnp.ones((1, D), f32))
    seq_id = jnp.arange(TH, dtype=jnp.int32) // L
    bias = jnp.where(seq_id[:, None] == seq_id[None, :], 0.0, _NEG)
    bias = bias.astype(f32)
    onesbd = jnp.kron(jnp.eye(HT, dtype=jnp.bfloat16),
                      jnp.ones((TH, 1), jnp.bfloat16))

    row = lambda a: a.reshape(1, -1)
    grid1 = SEQ // SBLK
    const = lambda shape: pl.BlockSpec(shape, lambda i: (0,) * len(shape))

    distill = pl.pallas_call(
        _block_kernel,
        grid=(grid1,),
        in_specs=[
            pl.BlockSpec((SBLK, L, D), lambda i: (i, 0, 0)),
            const((TH, TH)), const((HT * TH, HT)),
            const((D, D)), const((D, D)), const((D, D)),
            const((1, D)), const((1, D)), const((1, D)),
            const((D, D)), const((1, D)),
            const((1, D)), const((1, D)),
            const((D, E)), const((1, E)),
            const((D, E * D)), const((1, E * D)),
            const((E * D, D)), const((E, D)),
            const((E, E * D)),
            const((1, D)), const((1, D)),
            const((D, D)), const((1, D)),
        ],
        out_specs=pl.BlockSpec((SBLK, L2, D), lambda i: (i, 0, 0)),
        out_shape=jax.ShapeDtypeStruct((SEQ, L2, D), f32),
    )(x3, bias, onesbd, wqs, Wk, Wv, row(bqs), row(bk), row(bv), Wo, row(bo),
      row(ln1_g), row(ln1_b), Wr, row(br), w1cat, b1cat, w2stack, be2,
      emat, row(ln2_g), row(ln2_b), Wd, row(bd))

    eye4 = jnp.eye(HN, dtype=f32)
    asrc_full = (eye4[:, None, :] * a_src[:, :, None]).reshape(OUT, HN)
    adst_full = (eye4[:, None, :] * a_dst[:, :, None]).reshape(OUT, HN)
    hidx = (jnp.arange(OUT, dtype=jnp.int32) // DHN)[None, :]
    hidx = jnp.broadcast_to(hidx, (N, OUT))
    m01 = (adj > 0).astype(jnp.bfloat16)
    onesind = jnp.kron(jnp.eye(HN, dtype=jnp.bfloat16),
                       jnp.ones((N, 1), jnp.bfloat16))

    out = pl.pallas_call(
        _gat_kernel,
        grid=(L2 // 2,),
        in_specs=[
            pl.BlockSpec((SEQ, 2, 1, D), lambda g: (0, g, 0, 0)),
            pl.BlockSpec((2, N, N), lambda g: (0, 0, 0)),
            pl.BlockSpec((D, OUT), lambda g: (0, 0)),
            pl.BlockSpec((OUT, HN), lambda g: (0, 0)),
            pl.BlockSpec((OUT, HN), lambda g: (0, 0)),
            pl.BlockSpec((N, OUT), lambda g: (0, 0)),
            pl.BlockSpec((HN * N, HN), lambda g: (0, 0)),
        ],
        out_specs=pl.BlockSpec((B, N, 2, 1, OUT),
                               lambda g: (0, 0, g, 0, 0)),
        out_shape=jax.ShapeDtypeStruct((B, N, L2, 1, OUT), f32),
    )(distill.reshape(SEQ, L2, 1, D), m01, Wg, asrc_full, adst_full, hidx,
      onesind)
    return out.reshape(B, N, L2, OUT)


# speedup vs baseline: 1.0077x; 1.0077x over previous
import functools

import jax
import jax.numpy as jnp
from jax.experimental import pallas as pl

B, N, L, D = 2, 512, 32, 64
E, K = 16, 2
HT, HN = 4, 4
OUT = 64
DH = D // HT
DHN = OUT // HN
L2 = L // 2
SEQ = B * N

SBLK = 16
TBLK = SBLK * L
SPLIT = 2
TH = TBLK // SPLIT

_NEG = -1e9


def _ln(z, g, b):
    m = jnp.mean(z, axis=-1, keepdims=True)
    v = jnp.mean((z - m) ** 2, axis=-1, keepdims=True)
    return (z - m) * jax.lax.rsqrt(v + 1e-5) * g + b


def _block_kernel(x_ref, bias_ref, onesbd_ref, wq_ref, wk_ref, wv_ref,
                  bq_ref, bk_ref, bv_ref, wo_ref, bo_ref, ln1g_ref, ln1b_ref,
                  wr_ref, br_ref, w1cat_ref, b1cat_ref, w2stack_ref, be2_ref,
                  emat_ref, ln2g_ref, ln2b_ref, wd_ref, bd_ref, out_ref):
    xb = x_ref[...].reshape(TBLK, D)
    f32 = jnp.float32
    bf = jnp.bfloat16
    bias = bias_ref[...]

    q = (jnp.dot(xb, wq_ref[...], preferred_element_type=f32)
         + bq_ref[...]).astype(bf)
    k = (jnp.dot(xb, wk_ref[...], preferred_element_type=f32)
         + bk_ref[...]).astype(bf)
    v = jnp.dot(xb, wv_ref[...], preferred_element_type=f32) + bv_ref[...]
    o_parts = []
    for part in range(SPLIT):
        r0 = part * TH
        heads = []
        ps = []
        for h in range(HT):
            qh = q[r0:r0 + TH, h * DH:(h + 1) * DH]
            kh = k[r0:r0 + TH, h * DH:(h + 1) * DH]
            vh = v[r0:r0 + TH, h * DH:(h + 1) * DH]
            s = jax.lax.dot_general(qh, kh, (((1,), (1,)), ((), ())),
                                    preferred_element_type=f32) + bias
            p = jnp.exp2(s)
            ps.append(p)
            heads.append(jnp.dot(p, vh, preferred_element_type=f32))
        sums = jnp.dot(jnp.concatenate(ps, axis=-1), onesbd_ref[...],
                       preferred_element_type=f32)
        inv = 1.0 / sums
        inv_cat = jnp.concatenate(
            [jnp.broadcast_to(inv[:, h:h + 1], (TH, DH)) for h in range(HT)],
            axis=-1)
        o_parts.append(jnp.concatenate(heads, axis=-1) * inv_cat)
    o = jnp.concatenate(o_parts, axis=0)
    o = jnp.dot(o, wo_ref[...], preferred_element_type=f32) + bo_ref[...]
    h1 = _ln(xb + o, ln1g_ref[...], ln1b_ref[...])

    logits = jnp.dot(h1, wr_ref[...], preferred_element_type=f32) + br_ref[...]
    eiota = jax.lax.broadcasted_iota(jnp.int32, (TBLK, E), 1)
    m1 = jnp.max(logits, axis=-1, keepdims=True)
    i1 = jnp.min(jnp.where(logits == m1, eiota, E), axis=-1, keepdims=True)
    lmask = jnp.where(eiota == i1, -jnp.inf, logits)
    m2 = jnp.max(lmask, axis=-1, keepdims=True)
    i2 = jnp.min(jnp.where(lmask == m2, eiota, E), axis=-1, keepdims=True)
    t = jnp.exp(m2 - m1)
    g1 = 1.0 / (1.0 + t)
    g2 = t / (1.0 + t)
    w16 = jnp.where(eiota == i1, g1, 0.0) + jnp.where(eiota == i2, g2, 0.0)

    hcat = jnp.maximum(
        jnp.dot(h1.astype(bf), w1cat_ref[...].astype(bf),
                preferred_element_type=f32) + b1cat_ref[...], 0.0)
    wexp = jnp.dot(w16.astype(bf), emat_ref[...].astype(bf),
                   preferred_element_type=f32)
    acc = (jnp.dot((hcat * wexp).astype(bf), w2stack_ref[...].astype(bf),
                   preferred_element_type=f32)
           + jnp.dot(w16, be2_ref[...], preferred_element_type=f32))

    h2 = _ln(h1 + acc, ln2g_ref[...], ln2b_ref[...])

    dz = jnp.dot(h2.astype(bf), wd_ref[...].astype(bf),
                 preferred_element_type=f32) + bd_ref[...]
    dz = jnp.maximum(dz, jnp.exp(jnp.minimum(dz, 0.0)) - 1.0)
    out_ref[...] = jnp.max(dz.reshape(TBLK // 2, 2, D), axis=1).reshape(
        SBLK, L2, D)


def _gat_kernel(tc_ref, mask_ref, wg_ref, asrc_ref, adst_ref, hidx_ref,
                ones_ref, out_ref):
    f32 = jnp.float32
    bf = jnp.bfloat16
    hidx = hidx_ref[...]
    zero = jnp.zeros((), bf)
    for l2s in range(2):
      mask = mask_ref[l2s]
      for bb in range(B):
        tc = tc_ref[bb * N:(bb + 1) * N, l2s, 0, :]
        hgf = jnp.dot(tc.astype(bf), wg_ref[...].astype(bf),
                      preferred_element_type=f32)
        hg = hgf.astype(bf)
        s4 = jnp.dot(hgf, asrc_ref[...], preferred_element_type=f32)
        d4 = jnp.dot(hgf, adst_ref[...], preferred_element_type=f32)
        d4t = d4.T
        es = jnp.exp(s4).astype(bf)
        es5 = jnp.exp(0.2 * s4).astype(bf)
        ed = jnp.exp(d4t).astype(bf)
        ed5 = jnp.exp(0.2 * d4t).astype(bf)
        alphas = []
        vstack = []
        for h in range(HN):
            s5c, ec = es5[:, h:h + 1], es[:, h:h + 1]
            edr, ed5r = ed[h:h + 1, :], ed5[h:h + 1, :]
            alphas.append(jnp.maximum(ec * edr, s5c * ed5r) * mask)
            vstack.append(jnp.where(hidx == h, hg, zero))
        p_cat = jnp.concatenate(alphas, axis=-1)
        v_stack = jnp.concatenate(vstack, axis=0)
        go = jnp.dot(p_cat, v_stack, preferred_element_type=f32)
        sums = jnp.dot(p_cat, ones_ref[...], preferred_element_type=f32)
        inv = 1.0 / sums
        inv_cat = jnp.concatenate(
            [jnp.broadcast_to(inv[:, h:h + 1], (N, DHN)) for h in range(HN)],
            axis=-1)
        out_ref[bb, :, l2s, 0, :] = jnp.maximum(go * inv_cat, 0.0)


@functools.partial(jax.jit, static_argnums=())
def kernel(x, adj, Wq, Wk, Wv, bq, bk, bv, Wo, bo, ln1_g, ln1_b, Wr, br,
           We1, be1, We2, be2, ln2_g, ln2_b, Wd, bd, Wg, a_src, a_dst):
    f32 = jnp.float32
    x3 = x.reshape(SEQ, L, D)
    scale = 1.4426950408889634 / (DH ** 0.5)
    wqs = Wq * scale
    bqs = bq * scale
    w1cat = We1.transpose(1, 0, 2).reshape(D, E * D)
    b1cat = be1.reshape(1, E * D)
    w2stack = We2.reshape(E * D, D)
    emat = jnp.kron(jnp.eye(E, dtype=f32), jnp.ones((1, D), f32))
    seq_id = jnp.arange(TH, dtype=jnp.int32) // L
    bias = jnp.where(seq_id[:, None] == seq_id[None, :], 0.0, _NEG)
    bias = bias.astype(f32)
    onesbd = jnp.kron(jnp.eye(HT, dtype=f32), jnp.ones((TH, 1), f32))

    row = lambda a: a.reshape(1, -1)
    grid1 = SEQ // SBLK
    const = lambda shape: pl.BlockSpec(shape, lambda i: (0,) * len(shape))

    distill = pl.pallas_call(
        _block_kernel,
        grid=(grid1,),
        in_specs=[
            pl.BlockSpec((SBLK, L, D), lambda i: (i, 0, 0)),
            const((TH, TH)), const((HT * TH, HT)),
            const((D, D)), const((D, D)), const((D, D)),
            const((1, D)), const((1, D)), const((1, D)),
            const((D, D)), const((1, D)),
            const((1, D)), const((1, D)),
            const((D, E)), const((1, E)),
            const((D, E * D)), const((1, E * D)),
            const((E * D, D)), const((E, D)),
            const((E, E * D)),
            const((1, D)), const((1, D)),
            const((D, D)), const((1, D)),
        ],
        out_specs=pl.BlockSpec((SBLK, L2, D), lambda i: (i, 0, 0)),
        out_shape=jax.ShapeDtypeStruct((SEQ, L2, D), f32),
    )(x3, bias, onesbd, wqs, Wk, Wv, row(bqs), row(bk), row(bv), Wo, row(bo),
      row(ln1_g), row(ln1_b), Wr, row(br), w1cat, b1cat, w2stack, be2,
      emat, row(ln2_g), row(ln2_b), Wd, row(bd))

    eye4 = jnp.eye(HN, dtype=f32)
    asrc_full = (eye4[:, None, :] * a_src[:, :, None]).reshape(OUT, HN)
    adst_full = (eye4[:, None, :] * a_dst[:, :, None]).reshape(OUT, HN)
    hidx = (jnp.arange(OUT, dtype=jnp.int32) // DHN)[None, :]
    hidx = jnp.broadcast_to(hidx, (N, OUT))
    m01 = (adj > 0).astype(jnp.bfloat16)
    onesind = jnp.kron(jnp.eye(HN, dtype=jnp.bfloat16),
                       jnp.ones((N, 1), jnp.bfloat16))

    out = pl.pallas_call(
        _gat_kernel,
        grid=(L2 // 2,),
        in_specs=[
            pl.BlockSpec((SEQ, 2, 1, D), lambda g: (0, g, 0, 0)),
            pl.BlockSpec((2, N, N), lambda g: (0, 0, 0)),
            pl.BlockSpec((D, OUT), lambda g: (0, 0)),
            pl.BlockSpec((OUT, HN), lambda g: (0, 0)),
            pl.BlockSpec((OUT, HN), lambda g: (0, 0)),
            pl.BlockSpec((N, OUT), lambda g: (0, 0)),
            pl.BlockSpec((HN * N, HN), lambda g: (0, 0)),
        ],
        out_specs=pl.BlockSpec((B, N, 2, 1, OUT),
                               lambda g: (0, 0, g, 0, 0)),
        out_shape=jax.ShapeDtypeStruct((B, N, L2, 1, OUT), f32),
    )(distill.reshape(SEQ, L2, 1, D), m01, Wg, asrc_full, adst_full, hidx,
      onesind)
    return out.reshape(B, N, L2, OUT)


# speedup vs baseline: 1.0157x; 1.0080x over previous
import functools

import jax
import jax.numpy as jnp
from jax.experimental import pallas as pl

B, N, L, D = 2, 512, 32, 64
E, K = 16, 2
HT, HN = 4, 4
OUT = 64
DH = D // HT
DHN = OUT // HN
L2 = L // 2
SEQ = B * N

SBLK = 16
TBLK = SBLK * L
SPLIT = 2
TH = TBLK // SPLIT

_NEG = -1e9


def _ln(z, g, b):
    m = jnp.mean(z, axis=-1, keepdims=True)
    v = jnp.mean((z - m) ** 2, axis=-1, keepdims=True)
    return (z - m) * jax.lax.rsqrt(v + 1e-5) * g + b


def _block_kernel(x_ref, bias_ref, onesbd_ref, wq_ref, wk_ref, wv_ref,
                  bq_ref, bk_ref, bv_ref, wo_ref, bo_ref, ln1g_ref, ln1b_ref,
                  wr_ref, br_ref, w1cat_ref, b1cat_ref, w2stack_ref, be2_ref,
                  emat_ref, ln2g_ref, ln2b_ref, wd_ref, bd_ref, out_ref):
    xb = x_ref[...].reshape(TBLK, D)
    f32 = jnp.float32
    bf = jnp.bfloat16
    bias = bias_ref[...]

    q = (jnp.dot(xb, wq_ref[...], preferred_element_type=f32)
         + bq_ref[...]).astype(bf)
    k = (jnp.dot(xb, wk_ref[...], preferred_element_type=f32)
         + bk_ref[...]).astype(bf)
    v = jnp.dot(xb, wv_ref[...], preferred_element_type=f32) + bv_ref[...]
    o_parts = []
    for part in range(SPLIT):
        r0 = part * TH
        heads = []
        ps = []
        for h in range(HT):
            qh = q[r0:r0 + TH, h * DH:(h + 1) * DH]
            kh = k[r0:r0 + TH, h * DH:(h + 1) * DH]
            vh = v[r0:r0 + TH, h * DH:(h + 1) * DH]
            s = jax.lax.dot_general(qh, kh, (((1,), (1,)), ((), ())),
                                    preferred_element_type=f32) + bias
            p = jnp.exp2(s)
            ps.append(p)
            heads.append(jnp.dot(p, vh, preferred_element_type=f32))
        sums = jnp.dot(jnp.concatenate(ps, axis=-1), onesbd_ref[...],
                       preferred_element_type=f32)
        inv = 1.0 / sums
        inv_cat = jnp.concatenate(
            [jnp.broadcast_to(inv[:, h:h + 1], (TH, DH)) for h in range(HT)],
            axis=-1)
        o_parts.append(jnp.concatenate(heads, axis=-1) * inv_cat)
    o = jnp.concatenate(o_parts, axis=0)
    o = jnp.dot(o, wo_ref[...], preferred_element_type=f32) + bo_ref[...]
    h1 = _ln(xb + o, ln1g_ref[...], ln1b_ref[...])

    logits = jnp.dot(h1, wr_ref[...], preferred_element_type=f32) + br_ref[...]
    eiota = jax.lax.broadcasted_iota(jnp.int32, (TBLK, E), 1)
    m1 = jnp.max(logits, axis=-1, keepdims=True)
    i1 = jnp.min(jnp.where(logits == m1, eiota, E), axis=-1, keepdims=True)
    lmask = jnp.where(eiota == i1, -jnp.inf, logits)
    m2 = jnp.max(lmask, axis=-1, keepdims=True)
    i2 = jnp.min(jnp.where(lmask == m2, eiota, E), axis=-1, keepdims=True)
    t = jnp.exp(m2 - m1)
    g1 = 1.0 / (1.0 + t)
    g2 = t / (1.0 + t)
    w16 = jnp.where(eiota == i1, g1, 0.0) + jnp.where(eiota == i2, g2, 0.0)

    hcat = jnp.maximum(
        jnp.dot(h1.astype(bf), w1cat_ref[...].astype(bf),
                preferred_element_type=f32) + b1cat_ref[...], 0.0)
    wexp = jnp.dot(w16.astype(bf), emat_ref[...].astype(bf),
                   preferred_element_type=f32)
    acc = (jnp.dot((hcat * wexp).astype(bf), w2stack_ref[...].astype(bf),
                   preferred_element_type=f32)
           + jnp.dot(w16, be2_ref[...], preferred_element_type=f32))

    h2 = _ln(h1 + acc, ln2g_ref[...], ln2b_ref[...])

    dz = jnp.dot(h2.astype(bf), wd_ref[...].astype(bf),
                 preferred_element_type=f32) + bd_ref[...]
    dz = jnp.maximum(dz, jnp.exp(jnp.minimum(dz, 0.0)) - 1.0)
    out_ref[...] = jnp.max(dz.reshape(TBLK // 2, 2, D), axis=1).reshape(
        SBLK, L2, D)


def _gat_kernel(tc_ref, mask_ref, wg_ref, asrc_ref, adst_ref, hmask_ref,
                ones_ref, out_ref):
    f32 = jnp.float32
    bf = jnp.bfloat16
    for l2s in range(2):
      mask = mask_ref[l2s]
      for bb in range(B):
        tc = tc_ref[bb * N:(bb + 1) * N, l2s, 0, :]
        hgf = jnp.dot(tc.astype(bf), wg_ref[...].astype(bf),
                      preferred_element_type=f32)
        hg = hgf.astype(bf)
        s4 = jnp.dot(hgf, asrc_ref[...], preferred_element_type=f32)
        d4 = jnp.dot(hgf, adst_ref[...], preferred_element_type=f32)
        d4t = d4.T
        es = jnp.exp(s4).astype(bf)
        es5 = jnp.exp(0.2 * s4).astype(bf)
        ed = jnp.exp(d4t).astype(bf)
        ed5 = jnp.exp(0.2 * d4t).astype(bf)
        alphas = []
        vstack = []
        for h in range(HN):
            s5c, ec = es5[:, h:h + 1], es[:, h:h + 1]
            edr, ed5r = ed[h:h + 1, :], ed5[h:h + 1, :]
            alphas.append(jnp.maximum(ec * edr, s5c * ed5r) * mask)
            vstack.append(hg * hmask_ref[h:h + 1, :])
        p_cat = jnp.concatenate(alphas, axis=-1)
        v_stack = jnp.concatenate(vstack, axis=0)
        go = jnp.dot(p_cat, v_stack, preferred_element_type=f32)
        sums = jnp.dot(p_cat, ones_ref[...], preferred_element_type=f32)
        inv = 1.0 / sums
        inv_cat = jnp.concatenate(
            [jnp.broadcast_to(inv[:, h:h + 1], (N, DHN)) for h in range(HN)],
            axis=-1)
        out_ref[bb, :, l2s, 0, :] = jnp.maximum(go * inv_cat, 0.0)


@functools.partial(jax.jit, static_argnums=())
def kernel(x, adj, Wq, Wk, Wv, bq, bk, bv, Wo, bo, ln1_g, ln1_b, Wr, br,
           We1, be1, We2, be2, ln2_g, ln2_b, Wd, bd, Wg, a_src, a_dst):
    f32 = jnp.float32
    x3 = x.reshape(SEQ, L, D)
    scale = 1.4426950408889634 / (DH ** 0.5)
    wqs = Wq * scale
    bqs = bq * scale
    w1cat = We1.transpose(1, 0, 2).reshape(D, E * D)
    b1cat = be1.reshape(1, E * D)
    w2stack = We2.reshape(E * D, D)
    emat = jnp.kron(jnp.eye(E, dtype=f32), jnp.ones((1, D), f32))
    seq_id = jnp.arange(TH, dtype=jnp.int32) // L
    bias = jnp.where(seq_id[:, None] == seq_id[None, :], 0.0, _NEG)
    bias = bias.astype(f32)
    onesbd = jnp.kron(jnp.eye(HT, dtype=f32), jnp.ones((TH, 1), f32))

    row = lambda a: a.reshape(1, -1)
    grid1 = SEQ // SBLK
    const = lambda shape: pl.BlockSpec(shape, lambda i: (0,) * len(shape))

    distill = pl.pallas_call(
        _block_kernel,
        grid=(grid1,),
        in_specs=[
            pl.BlockSpec((SBLK, L, D), lambda i: (i, 0, 0)),
            const((TH, TH)), const((HT * TH, HT)),
            const((D, D)), const((D, D)), const((D, D)),
            const((1, D)), const((1, D)), const((1, D)),
            const((D, D)), const((1, D)),
            const((1, D)), const((1, D)),
            const((D, E)), const((1, E)),
            const((D, E * D)), const((1, E * D)),
            const((E * D, D)), const((E, D)),
            const((E, E * D)),
            const((1, D)), const((1, D)),
            const((D, D)), const((1, D)),
        ],
        out_specs=pl.BlockSpec((SBLK, L2, D), lambda i: (i, 0, 0)),
        out_shape=jax.ShapeDtypeStruct((SEQ, L2, D), f32),
    )(x3, bias, onesbd, wqs, Wk, Wv, row(bqs), row(bk), row(bv), Wo, row(bo),
      row(ln1_g), row(ln1_b), Wr, row(br), w1cat, b1cat, w2stack, be2,
      emat, row(ln2_g), row(ln2_b), Wd, row(bd))

    eye4 = jnp.eye(HN, dtype=f32)
    asrc_full = (eye4[:, None, :] * a_src[:, :, None]).reshape(OUT, HN)
    adst_full = (eye4[:, None, :] * a_dst[:, :, None]).reshape(OUT, HN)
    hmask = jnp.kron(jnp.eye(HN, dtype=jnp.bfloat16),
                     jnp.ones((1, DHN), jnp.bfloat16))
    m01 = (adj > 0).astype(jnp.bfloat16)
    onesind = jnp.kron(jnp.eye(HN, dtype=jnp.bfloat16),
                       jnp.ones((N, 1), jnp.bfloat16))

    out = pl.pallas_call(
        _gat_kernel,
        grid=(L2 // 2,),
        in_specs=[
            pl.BlockSpec((SEQ, 2, 1, D), lambda g: (0, g, 0, 0)),
            pl.BlockSpec((2, N, N), lambda g: (0, 0, 0)),
            pl.BlockSpec((D, OUT), lambda g: (0, 0)),
            pl.BlockSpec((OUT, HN), lambda g: (0, 0)),
            pl.BlockSpec((OUT, HN), lambda g: (0, 0)),
            pl.BlockSpec((HN, OUT), lambda g: (0, 0)),
            pl.BlockSpec((HN * N, HN), lambda g: (0, 0)),
        ],
        out_specs=pl.BlockSpec((B, N, 2, 1, OUT),
                               lambda g: (0, 0, g, 0, 0)),
        out_shape=jax.ShapeDtypeStruct((B, N, L2, 1, OUT), f32),
    )(distill.reshape(SEQ, L2, 1, D), m01, Wg, asrc_full, adst_full, hmask,
      onesind)
    return out.reshape(B, N, L2, OUT)
